# baseline (speedup 1.0000x reference)
import functools

import jax
import jax.numpy as jnp
from jax import lax
from jax.experimental import pallas as pl
from jax.experimental.pallas import tpu as pltpu
from jax.experimental.pallas import tpu_sc as plsc

B, D, T = 16, 256, 576
K = 8192
N = B * T
TK = 512
NCHUNK = K // TK

NC, NS, L = 2, 16, 16
NW = NC * NS
TOK_W = N // NW
IDX_COLS = 96
IDX_ROWS_W = (N // IDX_COLS) // NW
CODES_W = K // NW


_BIGI = float(2.0 * K)
_INF = float("inf")


def _lex_less(va, ia, vb, ib):
    return (va < vb) | ((va == vb) & (ia < ib))


def _tc_dist_argmin(x_ref, es_ref, idx_ref, loss_ref):
    b = pl.program_id(0)
    xb = x_ref[0]
    x2 = jnp.sum(xb * xb, axis=0)
    rowf = lax.broadcasted_iota(jnp.int32, (TK, T), 0).astype(jnp.float32)
    v1 = jnp.full((T,), _INF, jnp.float32)
    i1 = jnp.full((T,), _BIGI, jnp.float32)
    v2 = jnp.full((T,), _INF, jnp.float32)
    i2 = jnp.full((T,), _BIGI, jnp.float32)
    for kk in range(NCHUNK):
        es = es_ref[pl.ds(kk * TK, TK), :]
        e2 = 0.25 * jnp.sum(es * es, axis=1)
        mm = lax.dot_general(es, xb, (((1,), (0,)), ((), ())),
                             preferred_element_type=jnp.float32)
        d2 = jnp.maximum((x2[None, :] + e2[:, None]) + mm, 0.0)
        mc = jnp.min(d2, axis=0)
        eq1 = d2 == mc[None, :]
        ic = jnp.min(jnp.where(eq1, rowf, _BIGI), axis=0) + (kk * TK)
        m2c = jnp.min(jnp.where(eq1, _INF, d2), axis=0)
        ic2 = jnp.min(jnp.where(d2 == m2c[None, :], rowf, _BIGI),
                      axis=0) + (kk * TK)
        for (vc, jc) in ((mc, ic), (m2c, ic2)):
            is1 = _lex_less(vc, jc, v1, i1)
            is2 = _lex_less(vc, jc, v2, i2)
            v2 = jnp.where(is1, v1, jnp.where(is2, vc, v2))
            i2 = jnp.where(is1, i1, jnp.where(is2, jc, i2))
            v1 = jnp.where(is1, vc, v1)
            i1 = jnp.where(is1, jc, i1)
    dist1 = jnp.sqrt(v1)
    dist2 = jnp.sqrt(v2)
    use2 = (dist2 < dist1) | ((dist2 == dist1) & (i2 < i1))
    besti = jnp.where(use2, i2, i1).astype(jnp.int32)
    idx_ref[0, 0, :] = besti
    part = jnp.sum(jnp.where(use2, v2, v1))

    @pl.when(b == 0)
    def _():
        loss_ref[...] = jnp.zeros_like(loss_ref)

    loss_ref[...] = loss_ref[...] + part

    @pl.when(b == B - 1)
    def _():
        loss_ref[...] = loss_ref[...] * (1.0 / (N * D))


_tc_call = pl.pallas_call(
    _tc_dist_argmin,
    grid=(B,),
    in_specs=[
        pl.BlockSpec((1, D, T), lambda b: (b, 0, 0)),
        pl.BlockSpec((K, D), lambda b: (0, 0)),
    ],
    out_specs=[
        pl.BlockSpec((1, 1, T), lambda b: (b, 0, 0)),
        pl.BlockSpec((1, 1), lambda b: (0, 0)),
    ],
    out_shape=[
        jax.ShapeDtypeStruct((B, 1, T), jnp.int32),
        jax.ShapeDtypeStruct((1, 1), jnp.float32),
    ],
)


def _sc_body(embed_hbm, idx_hbm, idxb_hbm, q_hbm, part_hbm,
             idx_v, rows_v, all_v, flags_v, stage_v, sem):
    w = lax.axis_index("s") * NC + lax.axis_index("c")

    pltpu.sync_copy(idx_hbm.at[w], idx_v)
    handles = []
    for j in range(IDX_ROWS_W):
        handles.append(pltpu.async_copy(
            embed_hbm.at[idx_v.at[j]],
            rows_v.at[pl.ds(j * IDX_COLS, IDX_COLS)], sem))
    for h in handles:
        h.wait()
    pltpu.sync_copy(rows_v, q_hbm.at[pl.ds(w * TOK_W, TOK_W)])

    pltpu.sync_copy(idxb_hbm, all_v)
    zeros16 = jnp.zeros((L,), jnp.float32)
    ones16 = jnp.ones((L,), jnp.float32)
    for c in range(CODES_W // L):
        flags_v[pl.ds(c * L, L)] = zeros16
    lo = w * CODES_W

    def row_body(r, carry):
        for c in range(128 // L):
            v = all_v[r, pl.ds(c * L, L)]
            msk = (v >= lo) & (v < lo + CODES_W)
            off = jnp.clip(v - lo, 0, CODES_W - 1)
            plsc.store_scatter(flags_v, [off], ones16, mask=msk)
        return carry

    lax.fori_loop(0, N // 128, row_body, 0)

    acc = jnp.zeros((L,), jnp.float32)
    for c in range(CODES_W // L):
        acc = acc + flags_v[pl.ds(c * L, L)]
    cnt = jnp.sum(acc)
    lane = lax.iota(jnp.int32, L)
    stage_v[...] = jnp.where(lane == 0, cnt, 0.0)
    pltpu.sync_copy(stage_v, part_hbm.at[pl.ds(w * L, L)])


@functools.lru_cache(maxsize=1)
def _get_sc_call():
    return pl.kernel(
        _sc_body,
        out_type=[
            jax.ShapeDtypeStruct((N, D), jnp.float32),
            jax.ShapeDtypeStruct((NW * L,), jnp.float32),
        ],
        mesh=plsc.VectorSubcoreMesh(core_axis_name="c", subcore_axis_name="s",
                                    num_cores=NC, num_subcores=NS),
        compiler_params=pltpu.CompilerParams(needs_layout_passes=False),
        scratch_types=[
            pltpu.VMEM((IDX_ROWS_W, IDX_COLS), jnp.int32),
            pltpu.VMEM((TOK_W, D), jnp.float32),
            pltpu.VMEM((N // 128, 128), jnp.int32),
            pltpu.VMEM((CODES_W,), jnp.float32),
            pltpu.VMEM((L,), jnp.float32),
            pltpu.SemaphoreType.DMA,
        ],
    )


def kernel(x, embed):
    idx16, loss = _tc_call(x, embed * jnp.float32(-2.0))
    idx_flat = idx16.reshape(N)
    idx2 = idx_flat.reshape(NW, IDX_ROWS_W, IDX_COLS)
    idxb = idx_flat.reshape(N // 128, 128)
    q, part = _get_sc_call()(embed, idx2, idxb)
    quantized_st = jnp.transpose(q.reshape(B, T, D), (0, 2, 1))
    util = jnp.sum(part) * (1.0 / K)
    return quantized_st, idx_flat, loss[0, 0], util


# speedup vs baseline: 1.3440x; 1.3440x over previous
import functools

import jax
import jax.numpy as jnp
from jax import lax
from jax.experimental import pallas as pl
from jax.experimental.pallas import tpu as pltpu
from jax.experimental.pallas import tpu_sc as plsc

B, D, T = 16, 256, 576
K = 8192
N = B * T
TK = 512
NCHUNK = K // TK

NC, NS, L = 2, 16, 16
NW = NC * NS
TOK_W = N // NW
IDX_COLS = 96
IDX_ROWS_W = (N // IDX_COLS) // NW
CODES_W = K // NW


_BIGI = float(2.0 * K)
_INF = float("inf")


def _lex_less(va, ia, vb, ib):
    return (va < vb) | ((va == vb) & (ia < ib))


_GR = 8


def _tc_dist_argmin(x_ref, es_ref, idx_ref, loss_ref):
    b = pl.program_id(0)
    xb = x_ref[0]
    x2 = jnp.sum(xb * xb, axis=0)
    subi = lax.broadcasted_iota(jnp.int32, (_GR, T), 0).astype(jnp.float32)
    V1 = jnp.full((_GR, T), _INF, jnp.float32)
    I1 = jnp.full((_GR, T), _BIGI, jnp.float32)
    V2 = jnp.full((_GR, T), _INF, jnp.float32)
    I2 = jnp.full((_GR, T), _BIGI, jnp.float32)
    for kk in range(NCHUNK):
        es = es_ref[pl.ds(kk * TK, TK), :]
        e2 = 0.25 * jnp.sum(es * es, axis=1)
        mm = lax.dot_general(es, xb, (((1,), (0,)), ((), ())),
                             preferred_element_type=jnp.float32)
        d2 = jnp.maximum((x2[None, :] + e2[:, None]) + mm, 0.0)
        for r in range(TK // _GR):
            v = d2[r * _GR:(r + 1) * _GR, :]
            ri = subi + float(kk * TK + r * _GR)
            is1 = v < V1
            disp = jnp.where(is1, V1, v)
            dispI = jnp.where(is1, I1, ri)
            I1 = jnp.where(is1, ri, I1)
            V1 = jnp.minimum(v, V1)
            is2 = disp < V2
            V2 = jnp.where(is2, disp, V2)
            I2 = jnp.where(is2, dispI, I2)
    h = _GR
    while h > 1:
        h //= 2
        av1, ai1, av2, ai2 = V1[:h], I1[:h], V2[:h], I2[:h]
        bv1, bi1, bv2, bi2 = V1[h:], I1[h:], V2[h:], I2[h:]
        lt = _lex_less(bv1, bi1, av1, ai1)
        nv1 = jnp.where(lt, bv1, av1)
        ni1 = jnp.where(lt, bi1, ai1)
        dv = jnp.where(lt, av1, bv1)
        di = jnp.where(lt, ai1, bi1)
        slt = _lex_less(av2, ai2, bv2, bi2)
        sv = jnp.where(slt, av2, bv2)
        si = jnp.where(slt, ai2, bi2)
        s2 = _lex_less(dv, di, sv, si)
        V1, I1 = nv1, ni1
        V2 = jnp.where(s2, dv, sv)
        I2 = jnp.where(s2, di, si)
    v1, i1, v2, i2 = V1[0], I1[0], V2[0], I2[0]
    dist1 = jnp.sqrt(v1)
    dist2 = jnp.sqrt(v2)
    use2 = (dist2 < dist1) | ((dist2 == dist1) & (i2 < i1))
    besti = jnp.where(use2, i2, i1).astype(jnp.int32)
    idx_ref[0, 0, :] = besti
    part = jnp.sum(jnp.where(use2, v2, v1))

    @pl.when(b == 0)
    def _():
        loss_ref[...] = jnp.zeros_like(loss_ref)

    loss_ref[...] = loss_ref[...] + part

    @pl.when(b == B - 1)
    def _():
        loss_ref[...] = loss_ref[...] * (1.0 / (N * D))


_tc_call = pl.pallas_call(
    _tc_dist_argmin,
    grid=(B,),
    in_specs=[
        pl.BlockSpec((1, D, T), lambda b: (b, 0, 0)),
        pl.BlockSpec((K, D), lambda b: (0, 0)),
    ],
    out_specs=[
        pl.BlockSpec((1, 1, T), lambda b: (b, 0, 0)),
        pl.BlockSpec((1, 1), lambda b: (0, 0)),
    ],
    out_shape=[
        jax.ShapeDtypeStruct((B, 1, T), jnp.int32),
        jax.ShapeDtypeStruct((1, 1), jnp.float32),
    ],
)


def _sc_body(embed_hbm, idx_hbm, idxb_hbm, q_hbm, part_hbm,
             idx_v, rows_v, all_v, flags_v, stage_v, sem):
    w = lax.axis_index("s") * NC + lax.axis_index("c")

    pltpu.sync_copy(idx_hbm.at[w], idx_v)
    handles = []
    for j in range(IDX_ROWS_W):
        handles.append(pltpu.async_copy(
            embed_hbm.at[idx_v.at[j]],
            rows_v.at[pl.ds(j * IDX_COLS, IDX_COLS)], sem))
    for h in handles:
        h.wait()
    pltpu.sync_copy(rows_v, q_hbm.at[pl.ds(w * TOK_W, TOK_W)])

    pltpu.sync_copy(idxb_hbm, all_v)
    zeros16 = jnp.zeros((L,), jnp.float32)
    ones16 = jnp.ones((L,), jnp.float32)
    for c in range(CODES_W // L):
        flags_v[pl.ds(c * L, L)] = zeros16
    lo = w * CODES_W

    def row_body(r, carry):
        for c in range(128 // L):
            v = all_v[r, pl.ds(c * L, L)]
            msk = (v >= lo) & (v < lo + CODES_W)
            off = jnp.clip(v - lo, 0, CODES_W - 1)
            plsc.store_scatter(flags_v, [off], ones16, mask=msk)
        return carry

    lax.fori_loop(0, N // 128, row_body, 0)

    acc = jnp.zeros((L,), jnp.float32)
    for c in range(CODES_W // L):
        acc = acc + flags_v[pl.ds(c * L, L)]
    cnt = jnp.sum(acc)
    lane = lax.iota(jnp.int32, L)
    stage_v[...] = jnp.where(lane == 0, cnt, 0.0)
    pltpu.sync_copy(stage_v, part_hbm.at[pl.ds(w * L, L)])


@functools.lru_cache(maxsize=1)
def _get_sc_call():
    return pl.kernel(
        _sc_body,
        out_type=[
            jax.ShapeDtypeStruct((N, D), jnp.float32),
            jax.ShapeDtypeStruct((NW * L,), jnp.float32),
        ],
        mesh=plsc.VectorSubcoreMesh(core_axis_name="c", subcore_axis_name="s",
                                    num_cores=NC, num_subcores=NS),
        compiler_params=pltpu.CompilerParams(needs_layout_passes=False),
        scratch_types=[
            pltpu.VMEM((IDX_ROWS_W, IDX_COLS), jnp.int32),
            pltpu.VMEM((TOK_W, D), jnp.float32),
            pltpu.VMEM((N // 128, 128), jnp.int32),
            pltpu.VMEM((CODES_W,), jnp.float32),
            pltpu.VMEM((L,), jnp.float32),
            pltpu.SemaphoreType.DMA,
        ],
    )


def kernel(x, embed):
    idx16, loss = _tc_call(x, embed * jnp.float32(-2.0))
    idx_flat = idx16.reshape(N)
    idx2 = idx_flat.reshape(NW, IDX_ROWS_W, IDX_COLS)
    idxb = idx_flat.reshape(N // 128, 128)
    q, part = _get_sc_call()(embed, idx2, idxb)
    quantized_st = jnp.transpose(q.reshape(B, T, D), (0, 2, 1))
    util = jnp.sum(part) * (1.0 / K)
    return quantized_st, idx_flat, loss[0, 0], util


# speedup vs baseline: 1.4008x; 1.0423x over previous
import functools

import jax
import jax.numpy as jnp
from jax import lax
from jax.experimental import pallas as pl
from jax.experimental.pallas import tpu as pltpu
from jax.experimental.pallas import tpu_sc as plsc

B, D, T = 16, 256, 576
K = 8192
N = B * T
TK = 512
NCHUNK = K // TK

NC, NS, L = 2, 16, 16
NW = NC * NS
TOK_W = N // NW
IDX_COLS = 96
IDX_ROWS_W = (N // IDX_COLS) // NW
CODES_W = K // NW


_BIGI = float(2.0 * K)
_INF = float("inf")


def _lex_less(va, ia, vb, ib):
    return (va < vb) | ((va == vb) & (ia < ib))


_GR = 8


def _tc_dist_argmin(x_ref, es_ref, idx_ref, loss_ref):
    b = pl.program_id(0)
    xb = x_ref[0]
    x2 = jnp.sum(xb * xb, axis=0)
    subi = lax.broadcasted_iota(jnp.int32, (_GR, T), 0).astype(jnp.float32)
    V1 = jnp.full((_GR, T), _INF, jnp.float32)
    I1 = jnp.full((_GR, T), _BIGI, jnp.float32)
    V2 = jnp.full((_GR, T), _INF, jnp.float32)
    I2 = jnp.full((_GR, T), _BIGI, jnp.float32)
    for kk in range(NCHUNK):
        es = es_ref[pl.ds(kk * TK, TK), :]
        e2 = 0.25 * jnp.sum(es * es, axis=1)
        mm = lax.dot_general(es, xb, (((1,), (0,)), ((), ())),
                             preferred_element_type=jnp.float32)
        d2 = (x2[None, :] + e2[:, None]) + mm
        for r in range(TK // _GR):
            v = d2[r * _GR:(r + 1) * _GR, :]
            ri = subi + float(kk * TK + r * _GR)
            is1 = v < V1
            disp = jnp.where(is1, V1, v)
            dispI = jnp.where(is1, I1, ri)
            I1 = jnp.where(is1, ri, I1)
            V1 = jnp.minimum(v, V1)
            is2 = disp < V2
            V2 = jnp.where(is2, disp, V2)
            I2 = jnp.where(is2, dispI, I2)
    h = _GR
    while h > 1:
        h //= 2
        av1, ai1, av2, ai2 = V1[:h], I1[:h], V2[:h], I2[:h]
        bv1, bi1, bv2, bi2 = V1[h:], I1[h:], V2[h:], I2[h:]
        lt = _lex_less(bv1, bi1, av1, ai1)
        nv1 = jnp.where(lt, bv1, av1)
        ni1 = jnp.where(lt, bi1, ai1)
        dv = jnp.where(lt, av1, bv1)
        di = jnp.where(lt, ai1, bi1)
        slt = _lex_less(av2, ai2, bv2, bi2)
        sv = jnp.where(slt, av2, bv2)
        si = jnp.where(slt, ai2, bi2)
        s2 = _lex_less(dv, di, sv, si)
        V1, I1 = nv1, ni1
        V2 = jnp.where(s2, dv, sv)
        I2 = jnp.where(s2, di, si)
    v1, i1, v2, i2 = V1[0], I1[0], V2[0], I2[0]
    v1c = jnp.maximum(v1, 0.0)
    v2c = jnp.maximum(v2, 0.0)
    dist1 = jnp.sqrt(v1c)
    dist2 = jnp.sqrt(v2c)
    use2 = (dist2 < dist1) | ((dist2 == dist1) & (i2 < i1))
    besti = jnp.where(use2, i2, i1).astype(jnp.int32)
    idx_ref[0, 0, :] = besti
    part = jnp.sum(jnp.where(use2, v2c, v1c))

    @pl.when(b == 0)
    def _():
        loss_ref[...] = jnp.zeros_like(loss_ref)

    loss_ref[...] = loss_ref[...] + part

    @pl.when(b == B - 1)
    def _():
        loss_ref[...] = loss_ref[...] * (1.0 / (N * D))


_tc_call = pl.pallas_call(
    _tc_dist_argmin,
    grid=(B,),
    in_specs=[
        pl.BlockSpec((1, D, T), lambda b: (b, 0, 0)),
        pl.BlockSpec((K, D), lambda b: (0, 0)),
    ],
    out_specs=[
        pl.BlockSpec((1, 1, T), lambda b: (b, 0, 0)),
        pl.BlockSpec((1, 1), lambda b: (0, 0)),
    ],
    out_shape=[
        jax.ShapeDtypeStruct((B, 1, T), jnp.int32),
        jax.ShapeDtypeStruct((1, 1), jnp.float32),
    ],
)


def _sc_body(embed_hbm, idx_hbm, idxb_hbm, q_hbm, part_hbm,
             idx_v, rows_v, all_v, flags_v, stage_v, sem):
    w = lax.axis_index("s") * NC + lax.axis_index("c")

    pltpu.sync_copy(idx_hbm.at[w], idx_v)
    handles = []
    for j in range(IDX_ROWS_W):
        handles.append(pltpu.async_copy(
            embed_hbm.at[idx_v.at[j]],
            rows_v.at[pl.ds(j * IDX_COLS, IDX_COLS)], sem))
    for h in handles:
        h.wait()
    pltpu.sync_copy(rows_v, q_hbm.at[pl.ds(w * TOK_W, TOK_W)])

    pltpu.sync_copy(idxb_hbm, all_v)
    zeros16 = jnp.zeros((L,), jnp.float32)
    ones16 = jnp.ones((L,), jnp.float32)
    for c in range(CODES_W // L):
        flags_v[pl.ds(c * L, L)] = zeros16
    lo = w * CODES_W

    def row_body(r, carry):
        for c in range(128 // L):
            v = all_v[r, pl.ds(c * L, L)]
            msk = (v >= lo) & (v < lo + CODES_W)
            off = jnp.clip(v - lo, 0, CODES_W - 1)
            plsc.store_scatter(flags_v, [off], ones16, mask=msk)
        return carry

    lax.fori_loop(0, N // 128, row_body, 0)

    acc = jnp.zeros((L,), jnp.float32)
    for c in range(CODES_W // L):
        acc = acc + flags_v[pl.ds(c * L, L)]
    cnt = jnp.sum(acc)
    lane = lax.iota(jnp.int32, L)
    stage_v[...] = jnp.where(lane == 0, cnt, 0.0)
    pltpu.sync_copy(stage_v, part_hbm.at[pl.ds(w * L, L)])


@functools.lru_cache(maxsize=1)
def _get_sc_call():
    return pl.kernel(
        _sc_body,
        out_type=[
            jax.ShapeDtypeStruct((N, D), jnp.float32),
            jax.ShapeDtypeStruct((NW * L,), jnp.float32),
        ],
        mesh=plsc.VectorSubcoreMesh(core_axis_name="c", subcore_axis_name="s",
                                    num_cores=NC, num_subcores=NS),
        compiler_params=pltpu.CompilerParams(needs_layout_passes=False),
        scratch_types=[
            pltpu.VMEM((IDX_ROWS_W, IDX_COLS), jnp.int32),
            pltpu.VMEM((TOK_W, D), jnp.float32),
            pltpu.VMEM((N // 128, 128), jnp.int32),
            pltpu.VMEM((CODES_W,), jnp.float32),
            pltpu.VMEM((L,), jnp.float32),
            pltpu.SemaphoreType.DMA,
        ],
    )


def kernel(x, embed):
    idx16, loss = _tc_call(x, embed * jnp.float32(-2.0))
    idx_flat = idx16.reshape(N)
    idx2 = idx_flat.reshape(NW, IDX_ROWS_W, IDX_COLS)
    idxb = idx_flat.reshape(N // 128, 128)
    q, part = _get_sc_call()(embed, idx2, idxb)
    quantized_st = jnp.transpose(q.reshape(B, T, D), (0, 2, 1))
    util = jnp.sum(part) * (1.0 / K)
    return quantized_st, idx_flat, loss[0, 0], util


# speedup vs baseline: 1.4354x; 1.0247x over previous
import functools

import jax
import jax.numpy as jnp
from jax import lax
from jax.experimental import pallas as pl
from jax.experimental.pallas import tpu as pltpu
from jax.experimental.pallas import tpu_sc as plsc

B, D, T = 16, 256, 576
K = 8192
N = B * T
TK = 512
NCHUNK = K // TK

NC, NS, L = 2, 16, 16
NW = NC * NS
TOK_W = N // NW
IDX_COLS = 96
IDX_ROWS_W = (N // IDX_COLS) // NW
CODES_W = K // NW


_BIGI = float(2.0 * K)
_INF = float("inf")


def _lex_less(va, ia, vb, ib):
    return (va < vb) | ((va == vb) & (ia < ib))


_GR = 8


def _tc_dist_argmin(x_ref, es_ref, idx_ref, loss_ref):
    b = pl.program_id(0)
    xb = x_ref[0]
    x2 = jnp.sum(xb * xb, axis=0)
    subi = lax.broadcasted_iota(jnp.int32, (_GR, T), 0).astype(jnp.float32)
    V1 = jnp.full((_GR, T), _INF, jnp.float32)
    I1 = jnp.full((_GR, T), _BIGI, jnp.float32)
    V2 = jnp.full((_GR, T), _INF, jnp.float32)
    I2 = jnp.full((_GR, T), _BIGI, jnp.float32)
    for kk in range(NCHUNK):
        es = es_ref[pl.ds(kk * TK, TK), :]
        e2 = 0.25 * jnp.sum(es * es, axis=1)
        mm = lax.dot_general(es, xb, (((1,), (0,)), ((), ())),
                             preferred_element_type=jnp.float32)
        d2 = (x2[None, :] + e2[:, None]) + mm
        for r in range(TK // _GR):
            v = d2[r * _GR:(r + 1) * _GR, :]
            base = float(kk * TK + r * _GR)
            is1 = v < V1
            disp = jnp.where(is1, V1, v)
            dispI = jnp.where(is1, I1, base)
            I1 = jnp.where(is1, base, I1)
            V1 = jnp.minimum(v, V1)
            is2 = disp < V2
            V2 = jnp.where(is2, disp, V2)
            I2 = jnp.where(is2, dispI, I2)
    I1 = I1 + subi
    I2 = I2 + subi
    h = _GR
    while h > 1:
        h //= 2
        av1, ai1, av2, ai2 = V1[:h], I1[:h], V2[:h], I2[:h]
        bv1, bi1, bv2, bi2 = V1[h:], I1[h:], V2[h:], I2[h:]
        lt = _lex_less(bv1, bi1, av1, ai1)
        nv1 = jnp.where(lt, bv1, av1)
        ni1 = jnp.where(lt, bi1, ai1)
        dv = jnp.where(lt, av1, bv1)
        di = jnp.where(lt, ai1, bi1)
        slt = _lex_less(av2, ai2, bv2, bi2)
        sv = jnp.where(slt, av2, bv2)
        si = jnp.where(slt, ai2, bi2)
        s2 = _lex_less(dv, di, sv, si)
        V1, I1 = nv1, ni1
        V2 = jnp.where(s2, dv, sv)
        I2 = jnp.where(s2, di, si)
    v1, i1, v2, i2 = V1[0], I1[0], V2[0], I2[0]
    v1c = jnp.maximum(v1, 0.0)
    v2c = jnp.maximum(v2, 0.0)
    dist1 = jnp.sqrt(v1c)
    dist2 = jnp.sqrt(v2c)
    use2 = (dist2 < dist1) | ((dist2 == dist1) & (i2 < i1))
    besti = jnp.where(use2, i2, i1).astype(jnp.int32)
    idx_ref[0, 0, :] = besti
    part = jnp.sum(jnp.where(use2, v2c, v1c))

    @pl.when(b == 0)
    def _():
        loss_ref[...] = jnp.zeros_like(loss_ref)

    loss_ref[...] = loss_ref[...] + part

    @pl.when(b == B - 1)
    def _():
        loss_ref[...] = loss_ref[...] * (1.0 / (N * D))


_tc_call = pl.pallas_call(
    _tc_dist_argmin,
    grid=(B,),
    in_specs=[
        pl.BlockSpec((1, D, T), lambda b: (b, 0, 0)),
        pl.BlockSpec((K, D), lambda b: (0, 0)),
    ],
    out_specs=[
        pl.BlockSpec((1, 1, T), lambda b: (b, 0, 0)),
        pl.BlockSpec((1, 1), lambda b: (0, 0)),
    ],
    out_shape=[
        jax.ShapeDtypeStruct((B, 1, T), jnp.int32),
        jax.ShapeDtypeStruct((1, 1), jnp.float32),
    ],
)


def _sc_body(embed_hbm, idx_hbm, idxb_hbm, q_hbm, part_hbm,
             idx_v, rows_v, all_v, flags_v, stage_v, sem):
    w = lax.axis_index("s") * NC + lax.axis_index("c")

    pltpu.sync_copy(idx_hbm.at[w], idx_v)
    handles = []
    for j in range(IDX_ROWS_W):
        handles.append(pltpu.async_copy(
            embed_hbm.at[idx_v.at[j]],
            rows_v.at[pl.ds(j * IDX_COLS, IDX_COLS)], sem))
    for h in handles:
        h.wait()
    pltpu.sync_copy(rows_v, q_hbm.at[pl.ds(w * TOK_W, TOK_W)])

    pltpu.sync_copy(idxb_hbm, all_v)
    zeros16 = jnp.zeros((L,), jnp.float32)
    ones16 = jnp.ones((L,), jnp.float32)
    for c in range(CODES_W // L):
        flags_v[pl.ds(c * L, L)] = zeros16
    lo = w * CODES_W

    def row_body(r, carry):
        for c in range(128 // L):
            v = all_v[r, pl.ds(c * L, L)]
            msk = (v >= lo) & (v < lo + CODES_W)
            off = jnp.clip(v - lo, 0, CODES_W - 1)
            plsc.store_scatter(flags_v, [off], ones16, mask=msk)
        return carry

    lax.fori_loop(0, N // 128, row_body, 0)

    acc = jnp.zeros((L,), jnp.float32)
    for c in range(CODES_W // L):
        acc = acc + flags_v[pl.ds(c * L, L)]
    cnt = jnp.sum(acc)
    lane = lax.iota(jnp.int32, L)
    stage_v[...] = jnp.where(lane == 0, cnt, 0.0)
    pltpu.sync_copy(stage_v, part_hbm.at[pl.ds(w * L, L)])


@functools.lru_cache(maxsize=1)
def _get_sc_call():
    return pl.kernel(
        _sc_body,
        out_type=[
            jax.ShapeDtypeStruct((N, D), jnp.float32),
            jax.ShapeDtypeStruct((NW * L,), jnp.float32),
        ],
        mesh=plsc.VectorSubcoreMesh(core_axis_name="c", subcore_axis_name="s",
                                    num_cores=NC, num_subcores=NS),
        compiler_params=pltpu.CompilerParams(needs_layout_passes=False),
        scratch_types=[
            pltpu.VMEM((IDX_ROWS_W, IDX_COLS), jnp.int32),
            pltpu.VMEM((TOK_W, D), jnp.float32),
            pltpu.VMEM((N // 128, 128), jnp.int32),
            pltpu.VMEM((CODES_W,), jnp.float32),
            pltpu.VMEM((L,), jnp.float32),
            pltpu.SemaphoreType.DMA,
        ],
    )


def kernel(x, embed):
    idx16, loss = _tc_call(x, embed * jnp.float32(-2.0))
    idx_flat = idx16.reshape(N)
    idx2 = idx_flat.reshape(NW, IDX_ROWS_W, IDX_COLS)
    idxb = idx_flat.reshape(N // 128, 128)
    q, part = _get_sc_call()(embed, idx2, idxb)
    quantized_st = jnp.transpose(q.reshape(B, T, D), (0, 2, 1))
    util = jnp.sum(part) * (1.0 / K)
    return quantized_st, idx_flat, loss[0, 0], util
